# baseline (speedup 1.0000x reference)
import functools

import jax
import jax.numpy as jnp
from jax import lax
from jax.experimental import pallas as pl
from jax.experimental.pallas import tpu as pltpu
from jax.experimental.pallas import tpu_sc as plsc

N = 50000
E = 1600000
G = 1000

NC = 2
NS = 16
CE = 128
NPAD = 50176
EP = 1605632
ROWS_PER_TILE = NPAD // NS
DUMMY_DST = 50100

_BN_SCALE = 1.0 / (1.0 + 1e-5) ** 0.5


def _tanh_vreg(z):
    ez = jnp.exp(z + z)
    return 1.0 - 2.0 / (ez + 1.0)


def _make_sc_conv(D, chunks_per_tile, edge_split):
    mesh = plsc.VectorSubcoreMesh(core_axis_name="c", subcore_axis_name="s",
                                  num_cores=NC, num_subcores=NS)
    assert chunks_per_tile % 2 == 0
    stage_rows = 112
    stage_chunks = ROWS_PER_TILE // stage_rows

    def body(tables, eap, src_h, dst_h, init_h, out_h,
             src0, src1, dst0, dst1, eap0, eap1, rows0, rows1, acc,
             isem0, isem1, dsem0, dsem1):
        c = lax.axis_index("c")
        s = lax.axis_index("s")
        srcb = (src0, src1)
        dstb = (dst0, dst1)
        eapb = (eap0, eap1)
        rowsb = (rows0, rows1)
        isems = (isem0, isem1)
        dsems = (dsem0, dsem1)

        def tbl_at(idx_ref):
            return (tables if edge_split else tables.at[c]).at[idx_ref]

        def eap_at(chunk):
            sl = pl.ds(chunk * CE, CE)
            return eap.at[sl] if edge_split else eap.at[c].at[sl]

        def init_chunk(i, carry):
            r = s * ROWS_PER_TILE + i * stage_rows
            pltpu.sync_copy(init_h.at[c].at[pl.ds(r, stage_rows)],
                            rows0.at[pl.ds(0, stage_rows)])
            pltpu.sync_copy(rows0.at[pl.ds(0, stage_rows)],
                            acc.at[pl.ds(r, stage_rows)])
            return carry
        lax.fori_loop(0, stage_chunks, init_chunk, 0)
        plsc.subcore_barrier()

        base0 = (c * NS + s if edge_split else s) * chunks_per_tile

        def fetch_idx(chunk, b):
            row = base0 + chunk
            pltpu.async_copy(src_h.at[row], srcb[b], isems[b])
            pltpu.async_copy(dst_h.at[row], dstb[b], isems[b])

        def drain_isem(b):
            pltpu.make_async_copy(src_h.at[0], srcb[b], isems[b]).wait()
            pltpu.make_async_copy(dst_h.at[0], dstb[b], isems[b]).wait()

        def fetch_data(chunk, b):
            pltpu.async_copy(eap_at(base0 + chunk), eapb[b], dsems[b])
            pltpu.async_copy(tbl_at(srcb[b]), rowsb[b], dsems[b])

        def drain_dsem(b):
            pltpu.make_async_copy(eap_at(0), eapb[b], dsems[b]).wait()
            pltpu.make_async_copy(eap_at(0), rowsb[b], dsems[b]).wait()

        def compute_scatter(b):
            rows_v = rowsb[b]
            eap_v = eapb[b]

            def cbody(i, carry):
                for r in range(8):
                    e = i * 8 + r
                    for dd in range(D // 16):
                        sl = pl.ds(dd * 16, 16)
                        rows_v[e, sl] = _tanh_vreg(rows_v[e, sl] + eap_v[e, sl])
                return carry
            lax.fori_loop(0, CE // 8, cbody, 0)
            pltpu.sync_copy(rows_v, acc.at[dstb[b]], add=True)

        fetch_idx(0, 0)
        fetch_idx(1, 1)
        drain_isem(0)
        fetch_data(0, 0)

        def step(kk, carry):
            c0 = 2 * kk
            drain_isem(1)
            fetch_data(c0 + 1, 1)
            drain_dsem(0)
            compute_scatter(0)

            @pl.when(c0 + 2 < chunks_per_tile)
            def _():
                fetch_idx(c0 + 2, 0)

            @pl.when(c0 + 2 < chunks_per_tile)
            def _():
                drain_isem(0)
                fetch_data(c0 + 2, 0)
            drain_dsem(1)
            compute_scatter(1)

            @pl.when(c0 + 3 < chunks_per_tile)
            def _():
                fetch_idx(c0 + 3, 1)
            return carry
        lax.fori_loop(0, chunks_per_tile // 2, step, 0)

        plsc.subcore_barrier()

        def out_chunk(i, carry):
            r = s * ROWS_PER_TILE + i * stage_rows
            pltpu.sync_copy(acc.at[pl.ds(r, stage_rows)],
                            rows0.at[pl.ds(0, stage_rows)])
            pltpu.sync_copy(rows0.at[pl.ds(0, stage_rows)],
                            out_h.at[c].at[pl.ds(r, stage_rows)])
            return carry
        lax.fori_loop(0, stage_chunks, out_chunk, 0)

    return pl.kernel(
        body,
        out_type=jax.ShapeDtypeStruct((NC, NPAD, D), jnp.float32),
        mesh=mesh,
        compiler_params=pltpu.CompilerParams(use_tc_tiling_on_sc=False),
        scratch_types=[
            pltpu.VMEM((CE,), jnp.int32),
            pltpu.VMEM((CE,), jnp.int32),
            pltpu.VMEM((CE,), jnp.int32),
            pltpu.VMEM((CE,), jnp.int32),
            pltpu.VMEM((CE, D), jnp.float32),
            pltpu.VMEM((CE, D), jnp.float32),
            pltpu.VMEM((CE, D), jnp.float32),
            pltpu.VMEM((CE, D), jnp.float32),
            pltpu.VMEM_SHARED((NPAD, D), jnp.float32),
            pltpu.SemaphoreType.DMA,
            pltpu.SemaphoreType.DMA,
            pltpu.SemaphoreType.DMA,
            pltpu.SemaphoreType.DMA,
        ],
    )


_sc_conv1 = _make_sc_conv(32, EP // (NS * CE), edge_split=False)
_sc_conv2 = _make_sc_conv(16, EP // (NC * NS * CE), edge_split=True)

GP = 1024
RC = 112
NPT = NPAD // NS


def _readout_body(gsh_h, bat_h, h2_h, emb_h, att_h,
                  bat_v, g_v, gex_v, h2_v, valn_v, vald_v, att_v,
                  dbuf, nbuf, numer_sh, den_sh):
    c = lax.axis_index("c")
    s = lax.axis_index("s")
    zero16 = jnp.zeros((16,), jnp.float32)
    zidx = jnp.zeros((16,), jnp.int32)

    def z_row(r, carry):
        valn_v[r, pl.ds(0, 16)] = zero16
        return carry
    lax.fori_loop(0, GP // NS, z_row, 0)
    pltpu.sync_copy(valn_v.at[pl.ds(0, GP // NS)],
                    numer_sh.at[pl.ds(s * (GP // NS), GP // NS)])
    pltpu.sync_copy(valn_v.at[pl.ds(0, GP // NS)],
                    den_sh.at[pl.ds(s * (GP // NS), GP // NS)])
    plsc.subcore_barrier()

    def acc_chunk(i, carry):
        base = s * NPT + i * RC
        pltpu.sync_copy(gsh_h.at[pl.ds(base, RC)], g_v)
        pltpu.sync_copy(bat_h.at[pl.ds(base, RC)], bat_v)
        pltpu.sync_copy(h2_h.at[pl.ds(base, RC)], h2_v)

        def vexp(j, carry2):
            gex_v[pl.ds(j * 16, 16)] = jnp.exp(g_v[pl.ds(j * 16, 16)])
            return carry2
        lax.fori_loop(0, RC // 16, vexp, 0)

        def rowfill(j, carry2):
            ge = gex_v[pl.ds(j * 16, 16)]
            for r in range(16):
                e = j * 16 + r
                z = ge[r]
                valn_v[e, pl.ds(0, 16)] = h2_v[e, pl.ds(0, 16)] * z
                vald_v[e, pl.ds(0, 16)] = jnp.full((16,), z, jnp.float32)
            return carry2
        lax.fori_loop(0, RC // 16, rowfill, 0)

        pltpu.sync_copy(valn_v, numer_sh.at[bat_v], add=True)
        pltpu.sync_copy(vald_v, den_sh.at[bat_v], add=True)
        return carry
    lax.fori_loop(0, NPT // RC, acc_chunk, 0)
    plsc.subcore_barrier()

    pltpu.sync_copy(den_sh, dbuf)

    def att_chunk(i, carry):
        base = c * (NPAD // 2) + s * (NPAD // 2 // NS) + i * RC
        pltpu.sync_copy(gsh_h.at[pl.ds(base, RC)], g_v)
        pltpu.sync_copy(bat_h.at[pl.ds(base, RC)], bat_v)

        def vatt(j, carry2):
            sl = pl.ds(j * 16, 16)
            ge = jnp.exp(g_v[sl])
            den = plsc.load_gather(dbuf, [bat_v[sl], zidx])
            att_v[sl] = ge / (den + 1e-16)
            return carry2
        lax.fori_loop(0, RC // 16, vatt, 0)
        pltpu.sync_copy(att_v, att_h.at[pl.ds(base, RC)])
        return carry
    lax.fori_loop(0, NPAD // 2 // NS // RC, att_chunk, 0)

    @pl.when(c == 0)
    def _():
        r0 = s * (GP // NS)
        pltpu.sync_copy(numer_sh.at[pl.ds(r0, GP // NS)], nbuf)

        def erow(r, carry):
            dvec = dbuf[r0 + r, pl.ds(0, 16)]
            dv = jnp.full((16,), dvec[0], jnp.float32)
            nbuf[r, pl.ds(0, 16)] = nbuf[r, pl.ds(0, 16)] / (dv + 1e-16)
            return carry
        lax.fori_loop(0, GP // NS, erow, 0)
        pltpu.sync_copy(nbuf, emb_h.at[pl.ds(r0, GP // NS)])


_sc_readout = pl.kernel(
    _readout_body,
    out_type=(jax.ShapeDtypeStruct((GP, 16), jnp.float32),
              jax.ShapeDtypeStruct((NPAD,), jnp.float32)),
    mesh=plsc.VectorSubcoreMesh(core_axis_name="c", subcore_axis_name="s",
                                num_cores=NC, num_subcores=NS),
    compiler_params=pltpu.CompilerParams(use_tc_tiling_on_sc=False,
                                         needs_layout_passes=False),
    scratch_types=[
        pltpu.VMEM((RC,), jnp.int32),
        pltpu.VMEM((RC,), jnp.float32),
        pltpu.VMEM((RC,), jnp.float32),
        pltpu.VMEM((RC, 16), jnp.float32),
        pltpu.VMEM((RC, 16), jnp.float32),
        pltpu.VMEM((RC, 16), jnp.float32),
        pltpu.VMEM((RC,), jnp.float32),
        pltpu.VMEM((GP, 16), jnp.float32),
        pltpu.VMEM((GP // NS, 16), jnp.float32),
        pltpu.VMEM_SHARED((GP, 16), jnp.float32),
        pltpu.VMEM_SHARED((GP, 16), jnp.float32),
    ],
)


NB = 1024
NBLK = NPAD // NB


def _prep1_body(x_ref, wx_ref, bx_ref, wr_ref, br_ref, xps_ref, init_ref):
    xb = x_ref[...]
    xps_ref[0] = xb @ wx_ref[0] + bx_ref[0]
    init_ref[0] = jnp.tanh(xb @ wr_ref[0] + br_ref[0])


def _prep1(xp40, Wx_s, bx_s, Wr_s, br_s):
    return pl.pallas_call(
        _prep1_body,
        grid=(2, NBLK),
        in_specs=[
            pl.BlockSpec((NB, 40), lambda h, i: (i, 0)),
            pl.BlockSpec((1, 40, 32), lambda h, i: (h, 0, 0)),
            pl.BlockSpec((1, 1, 32), lambda h, i: (h, 0, 0)),
            pl.BlockSpec((1, 40, 32), lambda h, i: (h, 0, 0)),
            pl.BlockSpec((1, 1, 32), lambda h, i: (h, 0, 0)),
        ],
        out_specs=[
            pl.BlockSpec((1, NB, 32), lambda h, i: (h, i, 0)),
            pl.BlockSpec((1, NB, 32), lambda h, i: (h, i, 0)),
        ],
        out_shape=[
            jax.ShapeDtypeStruct((2, NPAD, 32), jnp.float32),
            jax.ShapeDtypeStruct((2, NPAD, 32), jnp.float32),
        ],
    )(xp40, Wx_s, bx_s, Wr_s, br_s)


def _eap_body(ea_ref, w_ref, out_ref):
    out_ref[0] = ea_ref[...] @ w_ref[0]


def _eap(eap8, W_s, nhalf, dout):
    BE = 4096
    return pl.pallas_call(
        _eap_body,
        grid=(nhalf, EP // BE),
        in_specs=[
            pl.BlockSpec((BE, 8), lambda h, i: (i, 0)),
            pl.BlockSpec((1, 8, dout), lambda h, i: (h, 0, 0)),
        ],
        out_specs=pl.BlockSpec((1, BE, dout), lambda h, i: (h, i, 0)),
        out_shape=jax.ShapeDtypeStruct((nhalf, EP, dout), jnp.float32),
    )(eap8, W_s)


def _prep2_body(h_ref, wh_ref, bh_ref, wr_ref, br_ref, xp2_ref, root2_ref):
    hb = h_ref[...]
    xp2_ref[...] = hb @ wh_ref[...] + bh_ref[...]
    root2_ref[...] = jnp.tanh(hb @ wr_ref[...] + br_ref[...])


def _prep2(h1p, W2h, b2, Wr2, br2):
    return pl.pallas_call(
        _prep2_body,
        grid=(NBLK,),
        in_specs=[
            pl.BlockSpec((NB, 64), lambda i: (i, 0)),
            pl.BlockSpec((64, 16), lambda i: (0, 0)),
            pl.BlockSpec((1, 16), lambda i: (0, 0)),
            pl.BlockSpec((64, 16), lambda i: (0, 0)),
            pl.BlockSpec((1, 16), lambda i: (0, 0)),
        ],
        out_specs=[
            pl.BlockSpec((NB, 16), lambda i: (i, 0)),
            pl.BlockSpec((NB, 16), lambda i: (i, 0)),
        ],
        out_shape=[
            jax.ShapeDtypeStruct((NPAD, 16), jnp.float32),
            jax.ShapeDtypeStruct((NPAD, 16), jnp.float32),
        ],
    )(h1p, W2h, b2[None, :], Wr2, br2[None, :])


def _gate_body(h_ref, w1_ref, b1_ref, w2_ref, b2_ref, w3_ref, b3_ref,
               gate_ref, bmax_ref):
    g = jnp.maximum(h_ref[...] @ w1_ref[...] + b1_ref[...], 0.0)
    g = jnp.maximum(g @ w2_ref[...] + b2_ref[...], 0.0)
    g = g @ w3_ref[...] + b3_ref[...]
    gate_ref[...] = g
    rid = lax.broadcasted_iota(jnp.int32, (NB, 1), 0) + pl.program_id(0) * NB
    m = jnp.max(jnp.where(rid < N, g[:, :1], -jnp.inf))
    bmax_ref[...] = jnp.broadcast_to(m, (1, 1, 128))


def _gate(h2p, Wg1, bg1, Wg2, bg2, Wg3, bg3):
    return pl.pallas_call(
        _gate_body,
        grid=(NBLK,),
        in_specs=[
            pl.BlockSpec((NB, 16), lambda i: (i, 0)),
            pl.BlockSpec((16, 64), lambda i: (0, 0)),
            pl.BlockSpec((1, 64), lambda i: (0, 0)),
            pl.BlockSpec((64, 32), lambda i: (0, 0)),
            pl.BlockSpec((1, 32), lambda i: (0, 0)),
            pl.BlockSpec((32, 128), lambda i: (0, 0)),
            pl.BlockSpec((1, 128), lambda i: (0, 0)),
        ],
        out_specs=[
            pl.BlockSpec((NB, 128), lambda i: (i, 0)),
            pl.BlockSpec((1, 1, 128), lambda i: (i, 0, 0)),
        ],
        out_shape=[
            jax.ShapeDtypeStruct((NPAD, 128), jnp.float32),
            jax.ShapeDtypeStruct((NBLK, 1, 128), jnp.float32),
        ],
    )(h2p, Wg1, bg1[None, :], Wg2, bg2[None, :], Wg3, bg3[None, :])


def _head_body(emb_ref, W1_ref, b1_ref, W2_ref, b2_ref, W3_ref, b3_ref,
               Wo_ref, bo_ref, s1_ref, t1_ref, s2_ref, t2_ref, s3_ref, t3_ref,
               out_ref):
    o = jnp.maximum(emb_ref[...] @ W1_ref[...] + b1_ref[...], 0.0)
    o = o * s1_ref[...] + t1_ref[...]
    o = jnp.maximum(o @ W2_ref[...] + b2_ref[...], 0.0)
    o = o * s2_ref[...] + t2_ref[...]
    o = jnp.maximum(o @ W3_ref[...] + b3_ref[...], 0.0)
    o = o * s3_ref[...] + t3_ref[...]
    out_ref[...] = o @ Wo_ref[...] + bo_ref[...]


def _mlp_head(emb, W1, b1, W2, b2, W3, b3, Wo, bo, g1, be1, g2, be2, g3, be3):
    Gp = 1024
    embp = jnp.zeros((Gp, 16), jnp.float32).at[:G].set(emb)
    out = pl.pallas_call(
        _head_body,
        out_shape=jax.ShapeDtypeStruct((Gp, 1), jnp.float32),
    )(embp, W1, b1[None, :], W2, b2[None, :], W3, b3[None, :], Wo, bo[None, :],
      (g1 * _BN_SCALE)[None, :], be1[None, :],
      (g2 * _BN_SCALE)[None, :], be2[None, :],
      (g3 * _BN_SCALE)[None, :], be3[None, :])
    return out[:G, 0]


def kernel(x, edge_index, edge_attr, batch, W_neg1, b_neg1, W_root1, b_root1,
           W_neg2, b_neg2, W_root2, b_root2, Wg1, bg1, Wg2, bg2, Wg3, bg3,
           W1, b1, W2, b2, W3, b3, Wo, bo, g1, be1, g2, be2, g3, be3):
    src = edge_index[1]
    dst = edge_index[0]
    srcp = jnp.pad(src, (0, EP - E)).reshape(EP // 128, 128)
    dstp = jnp.pad(dst, (0, EP - E), constant_values=DUMMY_DST).reshape(EP // 128, 128)
    eap8 = jnp.pad(edge_attr, ((0, EP - E), (0, 2)))

    W1x = W_neg1[:39]
    W1e = W_neg1[39:]
    xpad = jnp.pad(x, ((0, NPAD - N), (0, 1)))
    Wx_s = jnp.pad(jnp.stack([W1x[:, :32], W1x[:, 32:]]), ((0, 0), (0, 1), (0, 0)))
    bx_s = jnp.stack([b_neg1[None, :32], b_neg1[None, 32:]])
    Wr_s = jnp.pad(jnp.stack([W_root1[:, :32], W_root1[:, 32:]]),
                   ((0, 0), (0, 1), (0, 0)))
    br_s = jnp.stack([b_root1[None, :32], b_root1[None, 32:]])
    xps, init1 = _prep1(xpad, Wx_s, bx_s, Wr_s, br_s)
    W1e_s = jnp.pad(jnp.stack([W1e[:, :32], W1e[:, 32:]]), ((0, 0), (0, 2), (0, 0)))
    eaps1 = _eap(eap8, W1e_s, 2, 32)
    out1 = _sc_conv1(xps, eaps1, srcp, dstp, init1)
    h1p = jnp.concatenate([out1[0], out1[1]], axis=1)

    xp2, root2p = _prep2(h1p, W_neg2[:64], b_neg2, W_root2, b_root2)
    W2e_s = jnp.pad(W_neg2[64:], ((0, 2), (0, 0)))[None]
    eap2 = _eap(eap8, W2e_s, 1, 16)[0]
    init2 = jnp.stack([root2p, jnp.zeros_like(root2p)])
    out2 = _sc_conv2(xp2, eap2, srcp, dstp, init2)
    h2p = out2[0] + out2[1]

    Wg3p = jnp.pad(Wg3, ((0, 0), (0, 127)))
    bg3p = jnp.pad(bg3, (0, 127))
    gate_full, bmax = _gate(h2p, Wg1, bg1, Wg2, bg2, Wg3p, bg3p)
    gsh = gate_full[:, 0] - jnp.max(bmax)
    batp = jnp.pad(batch, (0, NPAD - N), constant_values=G)
    embf, attf = _sc_readout(gsh, batp, h2p)
    emb = embf[:G]
    att = attf[:N, None]

    o = _mlp_head(emb, W1, b1, W2, b2, W3, b3, Wo, bo,
                  g1, be1, g2, be2, g3, be3)
    return (o, att)


# speedup vs baseline: 1.0136x; 1.0136x over previous
import functools

import jax
import jax.numpy as jnp
from jax import lax
from jax.experimental import pallas as pl
from jax.experimental.pallas import tpu as pltpu
from jax.experimental.pallas import tpu_sc as plsc

N = 50000
E = 1600000
G = 1000

NC = 2
NS = 16
CE = 128
NPAD = 50176
EP = 1605632
ROWS_PER_TILE = NPAD // NS
DUMMY_DST = 50100

_BN_SCALE = 1.0 / (1.0 + 1e-5) ** 0.5


def _tanh_vreg(z):
    ez = jnp.exp(z + z)
    return 1.0 - 2.0 / (ez + 1.0)


def _make_sc_conv(D, chunks_per_tile, edge_split):
    mesh = plsc.VectorSubcoreMesh(core_axis_name="c", subcore_axis_name="s",
                                  num_cores=NC, num_subcores=NS)
    assert chunks_per_tile % 2 == 0
    stage_rows = 112
    stage_chunks = ROWS_PER_TILE // stage_rows

    def body(tables, eap, src_h, dst_h, init_h, out_h,
             src0, src1, dst0, dst1, eap0, eap1, rows0, rows1, acc,
             isem0, isem1, dsem0, dsem1):
        c = lax.axis_index("c")
        s = lax.axis_index("s")
        srcb = (src0, src1)
        dstb = (dst0, dst1)
        eapb = (eap0, eap1)
        rowsb = (rows0, rows1)
        isems = (isem0, isem1)
        dsems = (dsem0, dsem1)

        def tbl_at(idx_ref):
            return (tables if edge_split else tables.at[c]).at[idx_ref]

        def eap_at(chunk):
            sl = pl.ds(chunk * CE, CE)
            return eap.at[sl] if edge_split else eap.at[c].at[sl]

        def init_chunk(i, carry):
            r = s * ROWS_PER_TILE + i * stage_rows
            pltpu.sync_copy(init_h.at[c].at[pl.ds(r, stage_rows)],
                            rows0.at[pl.ds(0, stage_rows)])
            pltpu.sync_copy(rows0.at[pl.ds(0, stage_rows)],
                            acc.at[pl.ds(r, stage_rows)])
            return carry
        lax.fori_loop(0, stage_chunks, init_chunk, 0)
        plsc.subcore_barrier()

        base0 = (c * NS + s if edge_split else s) * chunks_per_tile

        def fetch_idx(chunk, b):
            row = base0 + chunk
            pltpu.async_copy(src_h.at[row], srcb[b], isems[b])
            pltpu.async_copy(dst_h.at[row], dstb[b], isems[b])

        def drain_isem(b):
            pltpu.make_async_copy(src_h.at[0], srcb[b], isems[b]).wait()
            pltpu.make_async_copy(dst_h.at[0], dstb[b], isems[b]).wait()

        def fetch_data(chunk, b):
            pltpu.async_copy(eap_at(base0 + chunk), eapb[b], dsems[b])
            pltpu.async_copy(tbl_at(srcb[b]), rowsb[b], dsems[b])

        def drain_dsem(b):
            pltpu.make_async_copy(eap_at(0), eapb[b], dsems[b]).wait()
            pltpu.make_async_copy(eap_at(0), rowsb[b], dsems[b]).wait()

        def compute_scatter(b):
            rows_v = rowsb[b]
            eap_v = eapb[b]

            def cbody(i, carry):
                for r in range(8):
                    e = i * 8 + r
                    for dd in range(D // 16):
                        sl = pl.ds(dd * 16, 16)
                        rows_v[e, sl] = _tanh_vreg(rows_v[e, sl] + eap_v[e, sl])
                return carry
            lax.fori_loop(0, CE // 8, cbody, 0)
            pltpu.sync_copy(rows_v, acc.at[dstb[b]], add=True)

        fetch_idx(0, 0)
        fetch_idx(1, 1)
        drain_isem(0)
        fetch_data(0, 0)

        def step(kk, carry):
            c0 = 2 * kk
            drain_isem(1)
            fetch_data(c0 + 1, 1)
            drain_dsem(0)
            compute_scatter(0)

            @pl.when(c0 + 2 < chunks_per_tile)
            def _():
                fetch_idx(c0 + 2, 0)

            @pl.when(c0 + 2 < chunks_per_tile)
            def _():
                drain_isem(0)
                fetch_data(c0 + 2, 0)
            drain_dsem(1)
            compute_scatter(1)

            @pl.when(c0 + 3 < chunks_per_tile)
            def _():
                fetch_idx(c0 + 3, 1)
            return carry
        lax.fori_loop(0, chunks_per_tile // 2, step, 0)

        plsc.subcore_barrier()

        def out_chunk(i, carry):
            r = s * ROWS_PER_TILE + i * stage_rows
            pltpu.sync_copy(acc.at[pl.ds(r, stage_rows)],
                            rows0.at[pl.ds(0, stage_rows)])
            pltpu.sync_copy(rows0.at[pl.ds(0, stage_rows)],
                            out_h.at[c].at[pl.ds(r, stage_rows)])
            return carry
        lax.fori_loop(0, stage_chunks, out_chunk, 0)

    return pl.kernel(
        body,
        out_type=jax.ShapeDtypeStruct((NC, NPAD, D), jnp.float32),
        mesh=mesh,
        compiler_params=pltpu.CompilerParams(use_tc_tiling_on_sc=False),
        scratch_types=[
            pltpu.VMEM((CE,), jnp.int32),
            pltpu.VMEM((CE,), jnp.int32),
            pltpu.VMEM((CE,), jnp.int32),
            pltpu.VMEM((CE,), jnp.int32),
            pltpu.VMEM((CE, D), jnp.float32),
            pltpu.VMEM((CE, D), jnp.float32),
            pltpu.VMEM((CE, D), jnp.float32),
            pltpu.VMEM((CE, D), jnp.float32),
            pltpu.VMEM_SHARED((NPAD, D), jnp.float32),
            pltpu.SemaphoreType.DMA,
            pltpu.SemaphoreType.DMA,
            pltpu.SemaphoreType.DMA,
            pltpu.SemaphoreType.DMA,
        ],
    )


_sc_conv1 = _make_sc_conv(32, EP // (NS * CE), edge_split=False)
_sc_conv2 = _make_sc_conv(16, EP // (NC * NS * CE), edge_split=True)

GP = 1024
RC = 112
NPT = NPAD // NS


def _readout_body(gsh_h, bat_h, h2_h, emb_h, att_h,
                  bat_v, g_v, gex_v, h2_v, valn_v, vald_v, att_v,
                  dbuf, nbuf, numer_sh, den_sh):
    c = lax.axis_index("c")
    s = lax.axis_index("s")
    zero16 = jnp.zeros((16,), jnp.float32)
    zidx = jnp.zeros((16,), jnp.int32)

    def z_row(r, carry):
        valn_v[r, pl.ds(0, 16)] = zero16
        return carry
    lax.fori_loop(0, GP // NS, z_row, 0)
    pltpu.sync_copy(valn_v.at[pl.ds(0, GP // NS)],
                    numer_sh.at[pl.ds(s * (GP // NS), GP // NS)])
    pltpu.sync_copy(valn_v.at[pl.ds(0, GP // NS)],
                    den_sh.at[pl.ds(s * (GP // NS), GP // NS)])
    plsc.subcore_barrier()

    def acc_chunk(i, carry):
        base = s * NPT + i * RC
        pltpu.sync_copy(gsh_h.at[pl.ds(base, RC)], g_v)
        pltpu.sync_copy(bat_h.at[pl.ds(base, RC)], bat_v)
        pltpu.sync_copy(h2_h.at[pl.ds(base, RC)], h2_v)

        def vexp(j, carry2):
            gex_v[pl.ds(j * 16, 16)] = jnp.exp(g_v[pl.ds(j * 16, 16)])
            return carry2
        lax.fori_loop(0, RC // 16, vexp, 0)

        def rowfill(j, carry2):
            ge = gex_v[pl.ds(j * 16, 16)]
            for r in range(16):
                e = j * 16 + r
                z = ge[r]
                valn_v[e, pl.ds(0, 16)] = h2_v[e, pl.ds(0, 16)] * z
                vald_v[e, pl.ds(0, 16)] = jnp.full((16,), z, jnp.float32)
            return carry2
        lax.fori_loop(0, RC // 16, rowfill, 0)

        pltpu.sync_copy(valn_v, numer_sh.at[bat_v], add=True)
        pltpu.sync_copy(vald_v, den_sh.at[bat_v], add=True)
        return carry
    lax.fori_loop(0, NPT // RC, acc_chunk, 0)
    plsc.subcore_barrier()

    pltpu.sync_copy(den_sh, dbuf)

    def att_chunk(i, carry):
        base = c * (NPAD // 2) + s * (NPAD // 2 // NS) + i * RC
        pltpu.sync_copy(gsh_h.at[pl.ds(base, RC)], g_v)
        pltpu.sync_copy(bat_h.at[pl.ds(base, RC)], bat_v)

        def vatt(j, carry2):
            sl = pl.ds(j * 16, 16)
            ge = jnp.exp(g_v[sl])
            den = plsc.load_gather(dbuf, [bat_v[sl], zidx])
            att_v[sl] = ge / (den + 1e-16)
            return carry2
        lax.fori_loop(0, RC // 16, vatt, 0)
        pltpu.sync_copy(att_v, att_h.at[pl.ds(base, RC)])
        return carry
    lax.fori_loop(0, NPAD // 2 // NS // RC, att_chunk, 0)

    @pl.when(c == 0)
    def _():
        r0 = s * (GP // NS)
        pltpu.sync_copy(numer_sh.at[pl.ds(r0, GP // NS)], nbuf)

        def erow(r, carry):
            dvec = dbuf[r0 + r, pl.ds(0, 16)]
            dv = jnp.full((16,), dvec[0], jnp.float32)
            nbuf[r, pl.ds(0, 16)] = nbuf[r, pl.ds(0, 16)] / (dv + 1e-16)
            return carry
        lax.fori_loop(0, GP // NS, erow, 0)
        pltpu.sync_copy(nbuf, emb_h.at[pl.ds(r0, GP // NS)])


_sc_readout = pl.kernel(
    _readout_body,
    out_type=(jax.ShapeDtypeStruct((GP, 16), jnp.float32),
              jax.ShapeDtypeStruct((NPAD,), jnp.float32)),
    mesh=plsc.VectorSubcoreMesh(core_axis_name="c", subcore_axis_name="s",
                                num_cores=NC, num_subcores=NS),
    compiler_params=pltpu.CompilerParams(use_tc_tiling_on_sc=False,
                                         needs_layout_passes=False),
    scratch_types=[
        pltpu.VMEM((RC,), jnp.int32),
        pltpu.VMEM((RC,), jnp.float32),
        pltpu.VMEM((RC,), jnp.float32),
        pltpu.VMEM((RC, 16), jnp.float32),
        pltpu.VMEM((RC, 16), jnp.float32),
        pltpu.VMEM((RC, 16), jnp.float32),
        pltpu.VMEM((RC,), jnp.float32),
        pltpu.VMEM((GP, 16), jnp.float32),
        pltpu.VMEM((GP // NS, 16), jnp.float32),
        pltpu.VMEM_SHARED((GP, 16), jnp.float32),
        pltpu.VMEM_SHARED((GP, 16), jnp.float32),
    ],
)


NB = 1024
NBLK = NPAD // NB


def _prep1_body(x_ref, wx_ref, bx_ref, wr_ref, br_ref, xps_ref, init_ref):
    xb = x_ref[...]
    xps_ref[0] = xb @ wx_ref[0] + bx_ref[0]
    init_ref[0] = jnp.tanh(xb @ wr_ref[0] + br_ref[0])


def _prep1(xp40, Wx_s, bx_s, Wr_s, br_s):
    return pl.pallas_call(
        _prep1_body,
        grid=(2, NBLK),
        in_specs=[
            pl.BlockSpec((NB, 40), lambda h, i: (i, 0)),
            pl.BlockSpec((1, 40, 32), lambda h, i: (h, 0, 0)),
            pl.BlockSpec((1, 1, 32), lambda h, i: (h, 0, 0)),
            pl.BlockSpec((1, 40, 32), lambda h, i: (h, 0, 0)),
            pl.BlockSpec((1, 1, 32), lambda h, i: (h, 0, 0)),
        ],
        out_specs=[
            pl.BlockSpec((1, NB, 32), lambda h, i: (h, i, 0)),
            pl.BlockSpec((1, NB, 32), lambda h, i: (h, i, 0)),
        ],
        out_shape=[
            jax.ShapeDtypeStruct((2, NPAD, 32), jnp.float32),
            jax.ShapeDtypeStruct((2, NPAD, 32), jnp.float32),
        ],
    )(xp40, Wx_s, bx_s, Wr_s, br_s)


def _eap_body(ea_ref, w_ref, out_ref):
    out_ref[0] = ea_ref[...] @ w_ref[0]


def _eap(eap8, W_s, nhalf, dout):
    BE = 16384
    return pl.pallas_call(
        _eap_body,
        grid=(nhalf, EP // BE),
        in_specs=[
            pl.BlockSpec((BE, 8), lambda h, i: (i, 0)),
            pl.BlockSpec((1, 8, dout), lambda h, i: (h, 0, 0)),
        ],
        out_specs=pl.BlockSpec((1, BE, dout), lambda h, i: (h, i, 0)),
        out_shape=jax.ShapeDtypeStruct((nhalf, EP, dout), jnp.float32),
    )(eap8, W_s)


def _prep2_body(h_ref, wh_ref, bh_ref, wr_ref, br_ref, xp2_ref, root2_ref):
    hb = h_ref[...]
    xp2_ref[...] = hb @ wh_ref[...] + bh_ref[...]
    root2_ref[...] = jnp.tanh(hb @ wr_ref[...] + br_ref[...])


def _prep2(h1p, W2h, b2, Wr2, br2):
    return pl.pallas_call(
        _prep2_body,
        grid=(NBLK,),
        in_specs=[
            pl.BlockSpec((NB, 64), lambda i: (i, 0)),
            pl.BlockSpec((64, 16), lambda i: (0, 0)),
            pl.BlockSpec((1, 16), lambda i: (0, 0)),
            pl.BlockSpec((64, 16), lambda i: (0, 0)),
            pl.BlockSpec((1, 16), lambda i: (0, 0)),
        ],
        out_specs=[
            pl.BlockSpec((NB, 16), lambda i: (i, 0)),
            pl.BlockSpec((NB, 16), lambda i: (i, 0)),
        ],
        out_shape=[
            jax.ShapeDtypeStruct((NPAD, 16), jnp.float32),
            jax.ShapeDtypeStruct((NPAD, 16), jnp.float32),
        ],
    )(h1p, W2h, b2[None, :], Wr2, br2[None, :])


def _gate_body(h_ref, w1_ref, b1_ref, w2_ref, b2_ref, w3_ref, b3_ref,
               gate_ref, bmax_ref):
    g = jnp.maximum(h_ref[...] @ w1_ref[...] + b1_ref[...], 0.0)
    g = jnp.maximum(g @ w2_ref[...] + b2_ref[...], 0.0)
    g = g @ w3_ref[...] + b3_ref[...]
    gate_ref[...] = g
    rid = lax.broadcasted_iota(jnp.int32, (NB, 1), 0) + pl.program_id(0) * NB
    m = jnp.max(jnp.where(rid < N, g[:, :1], -jnp.inf))
    bmax_ref[...] = jnp.broadcast_to(m, (1, 1, 128))


def _gate(h2p, Wg1, bg1, Wg2, bg2, Wg3, bg3):
    return pl.pallas_call(
        _gate_body,
        grid=(NBLK,),
        in_specs=[
            pl.BlockSpec((NB, 16), lambda i: (i, 0)),
            pl.BlockSpec((16, 64), lambda i: (0, 0)),
            pl.BlockSpec((1, 64), lambda i: (0, 0)),
            pl.BlockSpec((64, 32), lambda i: (0, 0)),
            pl.BlockSpec((1, 32), lambda i: (0, 0)),
            pl.BlockSpec((32, 128), lambda i: (0, 0)),
            pl.BlockSpec((1, 128), lambda i: (0, 0)),
        ],
        out_specs=[
            pl.BlockSpec((NB, 128), lambda i: (i, 0)),
            pl.BlockSpec((1, 1, 128), lambda i: (i, 0, 0)),
        ],
        out_shape=[
            jax.ShapeDtypeStruct((NPAD, 128), jnp.float32),
            jax.ShapeDtypeStruct((NBLK, 1, 128), jnp.float32),
        ],
    )(h2p, Wg1, bg1[None, :], Wg2, bg2[None, :], Wg3, bg3[None, :])


def _head_body(emb_ref, W1_ref, b1_ref, W2_ref, b2_ref, W3_ref, b3_ref,
               Wo_ref, bo_ref, s1_ref, t1_ref, s2_ref, t2_ref, s3_ref, t3_ref,
               out_ref):
    o = jnp.maximum(emb_ref[...] @ W1_ref[...] + b1_ref[...], 0.0)
    o = o * s1_ref[...] + t1_ref[...]
    o = jnp.maximum(o @ W2_ref[...] + b2_ref[...], 0.0)
    o = o * s2_ref[...] + t2_ref[...]
    o = jnp.maximum(o @ W3_ref[...] + b3_ref[...], 0.0)
    o = o * s3_ref[...] + t3_ref[...]
    out_ref[...] = o @ Wo_ref[...] + bo_ref[...]


def _mlp_head(emb, W1, b1, W2, b2, W3, b3, Wo, bo, g1, be1, g2, be2, g3, be3):
    Gp = 1024
    embp = jnp.zeros((Gp, 16), jnp.float32).at[:G].set(emb)
    out = pl.pallas_call(
        _head_body,
        out_shape=jax.ShapeDtypeStruct((Gp, 1), jnp.float32),
    )(embp, W1, b1[None, :], W2, b2[None, :], W3, b3[None, :], Wo, bo[None, :],
      (g1 * _BN_SCALE)[None, :], be1[None, :],
      (g2 * _BN_SCALE)[None, :], be2[None, :],
      (g3 * _BN_SCALE)[None, :], be3[None, :])
    return out[:G, 0]


def kernel(x, edge_index, edge_attr, batch, W_neg1, b_neg1, W_root1, b_root1,
           W_neg2, b_neg2, W_root2, b_root2, Wg1, bg1, Wg2, bg2, Wg3, bg3,
           W1, b1, W2, b2, W3, b3, Wo, bo, g1, be1, g2, be2, g3, be3):
    src = edge_index[1]
    dst = edge_index[0]
    srcp = jnp.pad(src, (0, EP - E)).reshape(EP // 128, 128)
    dstp = jnp.pad(dst, (0, EP - E), constant_values=DUMMY_DST).reshape(EP // 128, 128)
    eap8 = jnp.pad(edge_attr, ((0, EP - E), (0, 2)))

    W1x = W_neg1[:39]
    W1e = W_neg1[39:]
    xpad = jnp.pad(x, ((0, NPAD - N), (0, 1)))
    Wx_s = jnp.pad(jnp.stack([W1x[:, :32], W1x[:, 32:]]), ((0, 0), (0, 1), (0, 0)))
    bx_s = jnp.stack([b_neg1[None, :32], b_neg1[None, 32:]])
    Wr_s = jnp.pad(jnp.stack([W_root1[:, :32], W_root1[:, 32:]]),
                   ((0, 0), (0, 1), (0, 0)))
    br_s = jnp.stack([b_root1[None, :32], b_root1[None, 32:]])
    xps, init1 = _prep1(xpad, Wx_s, bx_s, Wr_s, br_s)
    W1e_s = jnp.pad(jnp.stack([W1e[:, :32], W1e[:, 32:]]), ((0, 0), (0, 2), (0, 0)))
    eaps1 = _eap(eap8, W1e_s, 2, 32)
    out1 = _sc_conv1(xps, eaps1, srcp, dstp, init1)
    h1p = jnp.concatenate([out1[0], out1[1]], axis=1)

    xp2, root2p = _prep2(h1p, W_neg2[:64], b_neg2, W_root2, b_root2)
    W2e_s = jnp.pad(W_neg2[64:], ((0, 2), (0, 0)))[None]
    eap2 = _eap(eap8, W2e_s, 1, 16)[0]
    init2 = jnp.stack([root2p, jnp.zeros_like(root2p)])
    out2 = _sc_conv2(xp2, eap2, srcp, dstp, init2)
    h2p = out2[0] + out2[1]

    Wg3p = jnp.pad(Wg3, ((0, 0), (0, 127)))
    bg3p = jnp.pad(bg3, (0, 127))
    gate_full, bmax = _gate(h2p, Wg1, bg1, Wg2, bg2, Wg3p, bg3p)
    gsh = gate_full[:, 0] - jnp.max(bmax)
    batp = jnp.pad(batch, (0, NPAD - N), constant_values=G)
    embf, attf = _sc_readout(gsh, batp, h2p)
    emb = embf[:G]
    att = attf[:N, None]

    o = _mlp_head(emb, W1, b1, W2, b2, W3, b3, Wo, bo,
                  g1, be1, g2, be2, g3, be3)
    return (o, att)


# speedup vs baseline: 1.3348x; 1.3169x over previous
import functools

import jax
import jax.numpy as jnp
from jax import lax
from jax.experimental import pallas as pl
from jax.experimental.pallas import tpu as pltpu
from jax.experimental.pallas import tpu_sc as plsc

N = 50000
E = 1600000
G = 1000

NC = 2
NS = 16
CE = 128
NPAD = 50176
EP = 1605632
ROWS_PER_TILE = NPAD // NS
DUMMY_DST = 50100

_BN_SCALE = 1.0 / (1.0 + 1e-5) ** 0.5


def _tanh_vreg(z):
    ez = jnp.exp(z + z)
    return 1.0 - 2.0 / (ez + 1.0)


def _make_sc_conv(D, chunks_per_tile, edge_split):
    mesh = plsc.VectorSubcoreMesh(core_axis_name="c", subcore_axis_name="s",
                                  num_cores=NC, num_subcores=NS)
    assert chunks_per_tile % 2 == 0
    stage_rows = 112
    stage_chunks = ROWS_PER_TILE // stage_rows

    def body(tables, eap, src_h, dst_h, init_h, out_h,
             src0, src1, dst0, dst1, eap0, eap1, rows0, rows1, acc,
             isem0, isem1, dsem0, dsem1):
        c = lax.axis_index("c")
        s = lax.axis_index("s")
        srcb = (src0, src1)
        dstb = (dst0, dst1)
        eapb = (eap0, eap1)
        rowsb = (rows0, rows1)
        isems = (isem0, isem1)
        dsems = (dsem0, dsem1)

        def tbl_at(idx_ref):
            return (tables if edge_split else tables.at[c]).at[idx_ref]

        def eap_at(chunk):
            sl = pl.ds(chunk * CE, CE)
            return eap.at[sl] if edge_split else eap.at[c].at[sl]

        def init_chunk(i, carry):
            r = s * ROWS_PER_TILE + i * stage_rows
            pltpu.sync_copy(init_h.at[c].at[pl.ds(r, stage_rows)],
                            rows0.at[pl.ds(0, stage_rows)])
            pltpu.sync_copy(rows0.at[pl.ds(0, stage_rows)],
                            acc.at[pl.ds(r, stage_rows)])
            return carry
        lax.fori_loop(0, stage_chunks, init_chunk, 0)
        plsc.subcore_barrier()

        base0 = (c * NS + s if edge_split else s) * chunks_per_tile

        def fetch_idx(chunk, b):
            row = base0 + chunk
            pltpu.async_copy(src_h.at[row], srcb[b], isems[b])
            pltpu.async_copy(dst_h.at[row], dstb[b], isems[b])

        def drain_isem(b):
            pltpu.make_async_copy(src_h.at[0], srcb[b], isems[b]).wait()
            pltpu.make_async_copy(dst_h.at[0], dstb[b], isems[b]).wait()

        def fetch_data(chunk, b):
            pltpu.async_copy(eap_at(base0 + chunk), eapb[b], dsems[b])
            pltpu.async_copy(tbl_at(srcb[b]), rowsb[b], dsems[b])

        def drain_dsem(b):
            pltpu.make_async_copy(eap_at(0), eapb[b], dsems[b]).wait()
            pltpu.make_async_copy(eap_at(0), rowsb[b], dsems[b]).wait()

        def compute_scatter(b):
            rows_v = rowsb[b]
            eap_v = eapb[b]

            def cbody(i, carry):
                for r in range(8):
                    e = i * 8 + r
                    for dd in range(D // 16):
                        sl = pl.ds(dd * 16, 16)
                        rows_v[e, sl] = _tanh_vreg(rows_v[e, sl] + eap_v[e, sl])
                return carry
            lax.fori_loop(0, CE // 8, cbody, 0)
            pltpu.sync_copy(rows_v, acc.at[dstb[b]], add=True)

        fetch_idx(0, 0)
        fetch_idx(1, 1)
        drain_isem(0)
        fetch_data(0, 0)

        def step(kk, carry):
            c0 = 2 * kk
            drain_isem(1)
            fetch_data(c0 + 1, 1)
            drain_dsem(0)
            compute_scatter(0)

            @pl.when(c0 + 2 < chunks_per_tile)
            def _():
                fetch_idx(c0 + 2, 0)

            @pl.when(c0 + 2 < chunks_per_tile)
            def _():
                drain_isem(0)
                fetch_data(c0 + 2, 0)
            drain_dsem(1)
            compute_scatter(1)

            @pl.when(c0 + 3 < chunks_per_tile)
            def _():
                fetch_idx(c0 + 3, 1)
            return carry
        lax.fori_loop(0, chunks_per_tile // 2, step, 0)

        plsc.subcore_barrier()

        def out_chunk(i, carry):
            r = s * ROWS_PER_TILE + i * stage_rows
            pltpu.sync_copy(acc.at[pl.ds(r, stage_rows)],
                            rows0.at[pl.ds(0, stage_rows)])
            pltpu.sync_copy(rows0.at[pl.ds(0, stage_rows)],
                            out_h.at[c].at[pl.ds(r, stage_rows)])
            return carry
        lax.fori_loop(0, stage_chunks, out_chunk, 0)

    return pl.kernel(
        body,
        out_type=jax.ShapeDtypeStruct((NC, NPAD, D), jnp.float32),
        mesh=mesh,
        compiler_params=pltpu.CompilerParams(use_tc_tiling_on_sc=False),
        scratch_types=[
            pltpu.VMEM((CE,), jnp.int32),
            pltpu.VMEM((CE,), jnp.int32),
            pltpu.VMEM((CE,), jnp.int32),
            pltpu.VMEM((CE,), jnp.int32),
            pltpu.VMEM((CE, D), jnp.float32),
            pltpu.VMEM((CE, D), jnp.float32),
            pltpu.VMEM((CE, D), jnp.float32),
            pltpu.VMEM((CE, D), jnp.float32),
            pltpu.VMEM_SHARED((NPAD, D), jnp.float32),
            pltpu.SemaphoreType.DMA,
            pltpu.SemaphoreType.DMA,
            pltpu.SemaphoreType.DMA,
            pltpu.SemaphoreType.DMA,
        ],
    )


_sc_conv1 = _make_sc_conv(32, EP // (NS * CE), edge_split=False)
_sc_conv2 = _make_sc_conv(16, EP // (NC * NS * CE), edge_split=True)

GP = 1024
RC = 112
NPT = NPAD // NS


def _readout_body(gsh_h, bat_h, h2_h, emb_h, att_h,
                  bat_v, g_v, gex_v, h2_v, valn_v, vald_v, att_v,
                  dbuf, nbuf, numer_sh, den_sh):
    c = lax.axis_index("c")
    s = lax.axis_index("s")
    zero16 = jnp.zeros((16,), jnp.float32)
    zidx = jnp.zeros((16,), jnp.int32)

    def z_row(r, carry):
        valn_v[r, pl.ds(0, 16)] = zero16
        return carry
    lax.fori_loop(0, GP // NS, z_row, 0)
    pltpu.sync_copy(valn_v.at[pl.ds(0, GP // NS)],
                    numer_sh.at[pl.ds(s * (GP // NS), GP // NS)])
    pltpu.sync_copy(valn_v.at[pl.ds(0, GP // NS)],
                    den_sh.at[pl.ds(s * (GP // NS), GP // NS)])
    plsc.subcore_barrier()

    def acc_chunk(i, carry):
        base = s * NPT + i * RC
        pltpu.sync_copy(gsh_h.at[pl.ds(base, RC)], g_v)
        pltpu.sync_copy(bat_h.at[pl.ds(base, RC)], bat_v)
        pltpu.sync_copy(h2_h.at[pl.ds(base, RC)], h2_v)

        def vexp(j, carry2):
            gex_v[pl.ds(j * 16, 16)] = jnp.exp(g_v[pl.ds(j * 16, 16)])
            return carry2
        lax.fori_loop(0, RC // 16, vexp, 0)

        def rowfill(j, carry2):
            ge = gex_v[pl.ds(j * 16, 16)]
            for r in range(16):
                e = j * 16 + r
                z = ge[r]
                valn_v[e, pl.ds(0, 16)] = h2_v[e, pl.ds(0, 16)] * z
                vald_v[e, pl.ds(0, 16)] = jnp.full((16,), z, jnp.float32)
            return carry2
        lax.fori_loop(0, RC // 16, rowfill, 0)

        pltpu.sync_copy(valn_v, numer_sh.at[bat_v], add=True)
        pltpu.sync_copy(vald_v, den_sh.at[bat_v], add=True)
        return carry
    lax.fori_loop(0, NPT // RC, acc_chunk, 0)
    plsc.subcore_barrier()

    pltpu.sync_copy(den_sh, dbuf)

    def att_chunk(i, carry):
        base = c * (NPAD // 2) + s * (NPAD // 2 // NS) + i * RC
        pltpu.sync_copy(gsh_h.at[pl.ds(base, RC)], g_v)
        pltpu.sync_copy(bat_h.at[pl.ds(base, RC)], bat_v)

        def vatt(j, carry2):
            sl = pl.ds(j * 16, 16)
            ge = jnp.exp(g_v[sl])
            den = plsc.load_gather(dbuf, [bat_v[sl], zidx])
            att_v[sl] = ge / (den + 1e-16)
            return carry2
        lax.fori_loop(0, RC // 16, vatt, 0)
        pltpu.sync_copy(att_v, att_h.at[pl.ds(base, RC)])
        return carry
    lax.fori_loop(0, NPAD // 2 // NS // RC, att_chunk, 0)

    @pl.when(c == 0)
    def _():
        r0 = s * (GP // NS)
        pltpu.sync_copy(numer_sh.at[pl.ds(r0, GP // NS)], nbuf)

        def erow(r, carry):
            dvec = dbuf[r0 + r, pl.ds(0, 16)]
            dv = jnp.full((16,), dvec[0], jnp.float32)
            nbuf[r, pl.ds(0, 16)] = nbuf[r, pl.ds(0, 16)] / (dv + 1e-16)
            return carry
        lax.fori_loop(0, GP // NS, erow, 0)
        pltpu.sync_copy(nbuf, emb_h.at[pl.ds(r0, GP // NS)])


_sc_readout = pl.kernel(
    _readout_body,
    out_type=(jax.ShapeDtypeStruct((GP, 16), jnp.float32),
              jax.ShapeDtypeStruct((NPAD,), jnp.float32)),
    mesh=plsc.VectorSubcoreMesh(core_axis_name="c", subcore_axis_name="s",
                                num_cores=NC, num_subcores=NS),
    compiler_params=pltpu.CompilerParams(use_tc_tiling_on_sc=False,
                                         needs_layout_passes=False),
    scratch_types=[
        pltpu.VMEM((RC,), jnp.int32),
        pltpu.VMEM((RC,), jnp.float32),
        pltpu.VMEM((RC,), jnp.float32),
        pltpu.VMEM((RC, 16), jnp.float32),
        pltpu.VMEM((RC, 16), jnp.float32),
        pltpu.VMEM((RC, 16), jnp.float32),
        pltpu.VMEM((RC,), jnp.float32),
        pltpu.VMEM((GP, 16), jnp.float32),
        pltpu.VMEM((GP // NS, 16), jnp.float32),
        pltpu.VMEM_SHARED((GP, 16), jnp.float32),
        pltpu.VMEM_SHARED((GP, 16), jnp.float32),
    ],
)


NB = 1024
NBLK = NPAD // NB


def _prep1_body(x_ref, wx_ref, bx_ref, wr_ref, br_ref, xps_ref, init_ref):
    xb = x_ref[...]
    xps_ref[0] = xb @ wx_ref[0] + bx_ref[0]
    init_ref[0] = jnp.tanh(xb @ wr_ref[0] + br_ref[0])


def _prep1(xp40, Wx_s, bx_s, Wr_s, br_s):
    return pl.pallas_call(
        _prep1_body,
        grid=(2, NBLK),
        in_specs=[
            pl.BlockSpec((NB, 40), lambda h, i: (i, 0)),
            pl.BlockSpec((1, 40, 32), lambda h, i: (h, 0, 0)),
            pl.BlockSpec((1, 1, 32), lambda h, i: (h, 0, 0)),
            pl.BlockSpec((1, 40, 32), lambda h, i: (h, 0, 0)),
            pl.BlockSpec((1, 1, 32), lambda h, i: (h, 0, 0)),
        ],
        out_specs=[
            pl.BlockSpec((1, NB, 32), lambda h, i: (h, i, 0)),
            pl.BlockSpec((1, NB, 32), lambda h, i: (h, i, 0)),
        ],
        out_shape=[
            jax.ShapeDtypeStruct((2, NPAD, 32), jnp.float32),
            jax.ShapeDtypeStruct((2, NPAD, 32), jnp.float32),
        ],
    )(xp40, Wx_s, bx_s, Wr_s, br_s)


def _eap_body(ea_ref, w_ref, out_ref):
    out_ref[0] = ea_ref[...] @ w_ref[0]


def _eap(eap8, W_s, nhalf, dout):
    BE = 16384
    return pl.pallas_call(
        _eap_body,
        grid=(nhalf, EP // BE),
        in_specs=[
            pl.BlockSpec((BE, 8), lambda h, i: (i, 0)),
            pl.BlockSpec((1, 8, dout), lambda h, i: (h, 0, 0)),
        ],
        out_specs=pl.BlockSpec((1, BE, dout), lambda h, i: (h, i, 0)),
        out_shape=jax.ShapeDtypeStruct((nhalf, EP, dout), jnp.float32),
    )(eap8, W_s)


def _prep2_body(h_ref, wh_ref, bh_ref, wr_ref, br_ref, xp2_ref, root2_ref):
    hb = h_ref[...]
    xp2_ref[...] = hb @ wh_ref[...] + bh_ref[...]
    root2_ref[...] = jnp.tanh(hb @ wr_ref[...] + br_ref[...])


def _prep2(h1p, W2h, b2, Wr2, br2):
    return pl.pallas_call(
        _prep2_body,
        grid=(NBLK,),
        in_specs=[
            pl.BlockSpec((NB, 64), lambda i: (i, 0)),
            pl.BlockSpec((64, 16), lambda i: (0, 0)),
            pl.BlockSpec((1, 16), lambda i: (0, 0)),
            pl.BlockSpec((64, 16), lambda i: (0, 0)),
            pl.BlockSpec((1, 16), lambda i: (0, 0)),
        ],
        out_specs=[
            pl.BlockSpec((NB, 16), lambda i: (i, 0)),
            pl.BlockSpec((NB, 16), lambda i: (i, 0)),
        ],
        out_shape=[
            jax.ShapeDtypeStruct((NPAD, 16), jnp.float32),
            jax.ShapeDtypeStruct((NPAD, 16), jnp.float32),
        ],
    )(h1p, W2h, b2[None, :], Wr2, br2[None, :])


def _gate_body(h_ref, w1_ref, b1_ref, w2_ref, b2_ref, w3_ref, b3_ref,
               gate_ref, bmax_ref):
    g = jnp.maximum(h_ref[...] @ w1_ref[...] + b1_ref[...], 0.0)
    g = jnp.maximum(g @ w2_ref[...] + b2_ref[...], 0.0)
    g = g @ w3_ref[...] + b3_ref[...]
    gate_ref[...] = g
    rid = lax.broadcasted_iota(jnp.int32, (NB, 1), 0) + pl.program_id(0) * NB
    m = jnp.max(jnp.where(rid < N, g[:, :1], -jnp.inf))
    bmax_ref[...] = jnp.broadcast_to(m, (1, 1, 128))


def _gate(h2p, Wg1, bg1, Wg2, bg2, Wg3, bg3):
    return pl.pallas_call(
        _gate_body,
        grid=(NBLK,),
        in_specs=[
            pl.BlockSpec((NB, 16), lambda i: (i, 0)),
            pl.BlockSpec((16, 64), lambda i: (0, 0)),
            pl.BlockSpec((1, 64), lambda i: (0, 0)),
            pl.BlockSpec((64, 32), lambda i: (0, 0)),
            pl.BlockSpec((1, 32), lambda i: (0, 0)),
            pl.BlockSpec((32, 128), lambda i: (0, 0)),
            pl.BlockSpec((1, 128), lambda i: (0, 0)),
        ],
        out_specs=[
            pl.BlockSpec((NB, 128), lambda i: (i, 0)),
            pl.BlockSpec((1, 1, 128), lambda i: (i, 0, 0)),
        ],
        out_shape=[
            jax.ShapeDtypeStruct((NPAD, 128), jnp.float32),
            jax.ShapeDtypeStruct((NBLK, 1, 128), jnp.float32),
        ],
    )(h2p, Wg1, bg1[None, :], Wg2, bg2[None, :], Wg3, bg3[None, :])


def _head_body(emb_ref, W1_ref, b1_ref, W2_ref, b2_ref, W3_ref, b3_ref,
               Wo_ref, bo_ref, s1_ref, t1_ref, s2_ref, t2_ref, s3_ref, t3_ref,
               out_ref):
    o = jnp.maximum(emb_ref[...] @ W1_ref[...] + b1_ref[...], 0.0)
    o = o * s1_ref[...] + t1_ref[...]
    o = jnp.maximum(o @ W2_ref[...] + b2_ref[...], 0.0)
    o = o * s2_ref[...] + t2_ref[...]
    o = jnp.maximum(o @ W3_ref[...] + b3_ref[...], 0.0)
    o = o * s3_ref[...] + t3_ref[...]
    out_ref[...] = o @ Wo_ref[...] + bo_ref[...]


def _mlp_head(emb, W1, b1, W2, b2, W3, b3, Wo, bo, g1, be1, g2, be2, g3, be3):
    Gp = 1024
    embp = jnp.zeros((Gp, 16), jnp.float32).at[:G].set(emb)
    out = pl.pallas_call(
        _head_body,
        out_shape=jax.ShapeDtypeStruct((Gp, 1), jnp.float32),
    )(embp, W1, b1[None, :], W2, b2[None, :], W3, b3[None, :], Wo, bo[None, :],
      (g1 * _BN_SCALE)[None, :], be1[None, :],
      (g2 * _BN_SCALE)[None, :], be2[None, :],
      (g3 * _BN_SCALE)[None, :], be3[None, :])
    return out[:G, 0]


def kernel(x, edge_index, edge_attr, batch, W_neg1, b_neg1, W_root1, b_root1,
           W_neg2, b_neg2, W_root2, b_root2, Wg1, bg1, Wg2, bg2, Wg3, bg3,
           W1, b1, W2, b2, W3, b3, Wo, bo, g1, be1, g2, be2, g3, be3):
    src = edge_index[1]
    dst = edge_index[0]
    srcp = jnp.pad(src, (0, EP - E)).reshape(EP // 128, 128)
    dstp = jnp.pad(dst, (0, EP - E), constant_values=DUMMY_DST).reshape(EP // 128, 128)
    eap8 = jnp.pad(edge_attr, ((0, EP - E), (0, 2)))

    W1x = W_neg1[:39]
    W1e = W_neg1[39:]
    xpad = jnp.pad(x, ((0, NPAD - N), (0, 1)))
    Wx_s = jnp.pad(jnp.stack([W1x[:, :32], W1x[:, 32:]]), ((0, 0), (0, 1), (0, 0)))
    bx_s = jnp.stack([b_neg1[None, :32], b_neg1[None, 32:]])
    Wr_s = jnp.pad(jnp.stack([W_root1[:, :32], W_root1[:, 32:]]),
                   ((0, 0), (0, 1), (0, 0)))
    br_s = jnp.stack([b_root1[None, :32], b_root1[None, 32:]])
    xps, init1 = _prep1(xpad, Wx_s, bx_s, Wr_s, br_s)
    W1e_s = jnp.pad(jnp.stack([W1e[:, :32], W1e[:, 32:]]), ((0, 0), (0, 2), (0, 0)))
    eaps1 = jnp.einsum('ek,hkd->hed', eap8, W1e_s)
    out1 = _sc_conv1(xps, eaps1, srcp, dstp, init1)
    h1p = jnp.concatenate([out1[0], out1[1]], axis=1)

    xp2, root2p = _prep2(h1p, W_neg2[:64], b_neg2, W_root2, b_root2)
    W2e_s = jnp.pad(W_neg2[64:], ((0, 2), (0, 0)))
    eap2 = eap8 @ W2e_s
    init2 = jnp.stack([root2p, jnp.zeros_like(root2p)])
    out2 = _sc_conv2(xp2, eap2, srcp, dstp, init2)
    h2p = out2[0] + out2[1]

    Wg3p = jnp.pad(Wg3, ((0, 0), (0, 127)))
    bg3p = jnp.pad(bg3, (0, 127))
    gate_full, bmax = _gate(h2p, Wg1, bg1, Wg2, bg2, Wg3p, bg3p)
    gsh = gate_full[:, 0] - jnp.max(bmax)
    batp = jnp.pad(batch, (0, NPAD - N), constant_values=G)
    embf, attf = _sc_readout(gsh, batp, h2p)
    emb = embf[:G]
    att = attf[:N, None]

    o = _mlp_head(emb, W1, b1, W2, b2, W3, b3, Wo, bo,
                  g1, be1, g2, be2, g3, be3)
    return (o, att)
